# Initial kernel scaffold; baseline (speedup 1.0000x reference)
import functools

import jax
import jax.numpy as jnp
from jax import lax
from jax.experimental import pallas as pl
from jax.experimental.pallas import tpu as pltpu
from jax.experimental.pallas import tpu_sc as plsc

P = 10
D = 128
N = 10000
E = 160000

NC = 2
NS = 16
NW = NC * NS

G_PER = 2 * E // NW
GB = 80
GC = G_PER // GB
GK = 5

S_PER = E // NW
SB = 40
SCH = S_PER // SB
SK = 5
ZROWS = N // NS

_MESH = dict(core_axis_name="c", subcore_axis_name="s")


def _gather_body(table, idx_hbm, out, idx_v, rows_v, sem_g, sem_s):
    wid = lax.axis_index("s") * NC + lax.axis_index("c")
    pltpu.sync_copy(idx_hbm.at[wid], idx_v)
    base = wid * G_PER

    def outer(j0, _):
        gets = []
        for b in range(GK):
            j = j0 * GK + b
            gets.append(pltpu.async_copy(table.at[idx_v.at[j]], rows_v.at[b], sem_g))
        puts = []
        for b in range(GK):
            gets[b].wait()
            j = j0 * GK + b
            puts.append(pltpu.async_copy(rows_v.at[b], out.at[pl.ds(base + j * GB, GB)], sem_s))
        for b in range(GK):
            puts[b].wait()
        return 0

    lax.fori_loop(0, GC // GK, outer, 0)


def _sc_gather(table, idx3):
    return pl.kernel(
        _gather_body,
        out_type=jax.ShapeDtypeStruct((2 * E, D), jnp.float32),
        mesh=plsc.VectorSubcoreMesh(**_MESH),
        scratch_types=[
            pltpu.VMEM((GC, GB), jnp.int32),
            pltpu.VMEM((GK, GB, D), jnp.float32),
            pltpu.SemaphoreType.DMA,
            pltpu.SemaphoreType.DMA,
        ],
    )(table, idx3)


def _scatter_body(e2, dst_hbm, zeros_hbm, out, dst_v, e_v, sem, agg_sh):
    c = lax.axis_index("c")
    s = lax.axis_index("s")
    wid = s * NC + c
    pltpu.sync_copy(zeros_hbm, agg_sh.at[pl.ds(s * ZROWS, ZROWS)])
    pltpu.sync_copy(dst_hbm.at[wid], dst_v)
    plsc.subcore_barrier()
    base = wid * S_PER

    def outer(j0, _):
        gets = []
        for b in range(SK):
            j = j0 * SK + b
            gets.append(pltpu.async_copy(e2.at[pl.ds(base + j * SB, SB)], e_v.at[b], sem))
        for b in range(SK):
            gets[b].wait()
            j = j0 * SK + b
            pltpu.sync_copy(e_v.at[b], agg_sh.at[dst_v.at[j]], add=True)
        return 0

    lax.fori_loop(0, SCH // SK, outer, 0)
    plsc.subcore_barrier()
    pltpu.sync_copy(agg_sh.at[pl.ds(s * ZROWS, ZROWS)],
                    out.at[c].at[pl.ds(s * ZROWS, ZROWS)])


def _sc_scatter(e2, dst3, zeros):
    return pl.kernel(
        _scatter_body,
        out_type=jax.ShapeDtypeStruct((NC, N, D), jnp.float32),
        mesh=plsc.VectorSubcoreMesh(**_MESH),
        scratch_types=[
            pltpu.VMEM((SCH, SB), jnp.int32),
            pltpu.VMEM((SK, SB, D), jnp.float32),
            pltpu.SemaphoreType.DMA,
            pltpu.VMEM_SHARED((N, D), jnp.float32),
        ],
    )(e2, dst3, zeros)


def _silu(v):
    return v * (1.0 / (1.0 + jnp.exp(-v)))


def _mlp_tail(h3, g, beta):
    mu = jnp.mean(h3, axis=-1, keepdims=True)
    dlt = h3 - mu
    var = jnp.mean(dlt * dlt, axis=-1, keepdims=True)
    return dlt * lax.rsqrt(var + 1e-5) * g + beta


def _edge_block(e_ref, gs_ref, gd_ref, w1e, w2, w3, b1, b2, b3, g, beta, out_ref):
    e = e_ref[...]
    pre = jnp.dot(e, w1e[...], preferred_element_type=jnp.float32)
    pre = pre + gs_ref[...] + gd_ref[...] + b1[...]
    h1 = _silu(pre)
    h2 = _silu(jnp.dot(h1, w2[...], preferred_element_type=jnp.float32) + b2[...])
    h3 = jnp.dot(h2, w3[...], preferred_element_type=jnp.float32) + b3[...]
    out_ref[...] = _mlp_tail(h3, g[...], beta[...]) + e


BE = 2000


def _tc_edge(e, gs, gd, w1e, w2, w3, b1, b2, b3, g, beta):
    row = lambda i: (i, 0)
    fix = lambda i: (0, 0)
    return pl.pallas_call(
        _edge_block,
        grid=(E // BE,),
        in_specs=[
            pl.BlockSpec((BE, D), row),
            pl.BlockSpec((BE, D), row),
            pl.BlockSpec((BE, D), row),
            pl.BlockSpec((D, D), fix),
            pl.BlockSpec((D, D), fix),
            pl.BlockSpec((D, D), fix),
            pl.BlockSpec((1, D), fix),
            pl.BlockSpec((1, D), fix),
            pl.BlockSpec((1, D), fix),
            pl.BlockSpec((1, D), fix),
            pl.BlockSpec((1, D), fix),
        ],
        out_specs=pl.BlockSpec((BE, D), row),
        out_shape=jax.ShapeDtypeStruct((E, D), jnp.float32),
    )(e, gs, gd, w1e, w2, w3, b1, b2, b3, g, beta)


def _node_block(agg2_ref, x_ref, w1a, w1x, w2, w3, b1, b2, b3, g, beta,
                wys, wyd, xo_ref, y_ref):
    x = x_ref[...]
    agg = agg2_ref[0] + agg2_ref[1]
    pre = (jnp.dot(agg, w1a[...], preferred_element_type=jnp.float32)
           + jnp.dot(x, w1x[...], preferred_element_type=jnp.float32) + b1[...])
    h1 = _silu(pre)
    h2 = _silu(jnp.dot(h1, w2[...], preferred_element_type=jnp.float32) + b2[...])
    h3 = jnp.dot(h2, w3[...], preferred_element_type=jnp.float32) + b3[...]
    xo = _mlp_tail(h3, g[...], beta[...]) + x
    xo_ref[...] = xo
    y_ref[0] = jnp.dot(xo, wys[...], preferred_element_type=jnp.float32)
    y_ref[1] = jnp.dot(xo, wyd[...], preferred_element_type=jnp.float32)


BN = 2000


def _tc_node(agg2, x, w1a, w1x, w2, w3, b1, b2, b3, g, beta, wys, wyd):
    row = lambda i: (i, 0)
    fix = lambda i: (0, 0)
    pair = lambda i: (0, i, 0)
    return pl.pallas_call(
        _node_block,
        grid=(N // BN,),
        in_specs=[
            pl.BlockSpec((2, BN, D), pair),
            pl.BlockSpec((BN, D), row),
            pl.BlockSpec((D, D), fix),
            pl.BlockSpec((D, D), fix),
            pl.BlockSpec((D, D), fix),
            pl.BlockSpec((D, D), fix),
            pl.BlockSpec((1, D), fix),
            pl.BlockSpec((1, D), fix),
            pl.BlockSpec((1, D), fix),
            pl.BlockSpec((1, D), fix),
            pl.BlockSpec((1, D), fix),
            pl.BlockSpec((D, D), fix),
            pl.BlockSpec((D, D), fix),
        ],
        out_specs=[pl.BlockSpec((BN, D), row), pl.BlockSpec((2, BN, D), pair)],
        out_shape=[jax.ShapeDtypeStruct((N, D), jnp.float32),
                   jax.ShapeDtypeStruct((2, N, D), jnp.float32)],
    )(agg2, x, w1a, w1x, w2, w3, b1, b2, b3, g, beta, wys, wyd)


def _init_block(x_ref, wys, wyd, y_ref):
    x = x_ref[...]
    y_ref[0] = jnp.dot(x, wys[...], preferred_element_type=jnp.float32)
    y_ref[1] = jnp.dot(x, wyd[...], preferred_element_type=jnp.float32)


def _tc_init(x, wys, wyd):
    return pl.pallas_call(
        _init_block,
        grid=(N // BN,),
        in_specs=[
            pl.BlockSpec((BN, D), lambda i: (i, 0)),
            pl.BlockSpec((D, D), lambda i: (0, 0)),
            pl.BlockSpec((D, D), lambda i: (0, 0)),
        ],
        out_specs=pl.BlockSpec((2, BN, D), lambda i: (0, i, 0)),
        out_shape=jax.ShapeDtypeStruct((2, N, D), jnp.float32),
    )(x, wys, wyd)


def kernel(node_features, edge_features, edge_index, eW1, eb1, eW2, eb2,
           eW3, eb3, eg, ebeta, nW1, nb1, nW2, nb2, nW3, nb3, ng, nbeta):
    src = edge_index[0]
    dst = edge_index[1]
    idx3 = jnp.concatenate([src, dst + N]).reshape(NW, GC, GB)
    dst3 = dst.reshape(NW, SCH, SB)
    zeros = jnp.zeros((ZROWS, D), jnp.float32)

    W1e = eW1[:, :D]
    W1s = eW1[:, D:2 * D]
    W1d = eW1[:, 2 * D:]
    nW1a = nW1[:, :D]
    nW1x = nW1[:, D:]
    r1 = lambda v: v.reshape(1, D)

    x = node_features
    e = edge_features
    y = _tc_init(x, W1s[0], W1d[0])
    for i in range(P):
        g2 = _sc_gather(y.reshape(2 * N, D), idx3)
        e = _tc_edge(e, g2[:E], g2[E:], W1e[i], eW2[i], eW3[i],
                     r1(eb1[i]), r1(eb2[i]), r1(eb3[i]), r1(eg[i]), r1(ebeta[i]))
        agg2 = _sc_scatter(e, dst3, zeros)
        j = min(i + 1, P - 1)
        x, y = _tc_node(agg2, x, nW1a[i], nW1x[i], nW2[i], nW3[i],
                        r1(nb1[i]), r1(nb2[i]), r1(nb3[i]), r1(ng[i]), r1(nbeta[i]),
                        W1s[j], W1d[j])
    return x


# speedup vs baseline: 3.0860x; 3.0860x over previous
import functools

import jax
import jax.numpy as jnp
from jax import lax
from jax.experimental import pallas as pl
from jax.experimental.pallas import tpu as pltpu
from jax.experimental.pallas import tpu_sc as plsc

P = 10
D = 128
N = 10000
E = 160000

NC = 2
NS = 16
NW = NC * NS

G_PER = 2 * E // NW
GB = 80
GC = G_PER // GB
GK = 5

S_PER = E // NW
SB = 40
SCH = S_PER // SB
SK = 5
ZB = 624
ZTAIL = N - NS * ZB

_MESH = dict(core_axis_name="c", subcore_axis_name="s")


def _gather_body(table, idx_hbm, out, idx_v, rows_v, sem_g, sem_s):
    wid = lax.axis_index("s") * NC + lax.axis_index("c")
    pltpu.sync_copy(idx_hbm.at[wid], idx_v)
    base = wid * G_PER

    def outer(j0, _):
        gets = []
        for b in range(GK):
            j = j0 * GK + b
            gets.append(pltpu.async_copy(table.at[idx_v.at[j]], rows_v.at[b], sem_g))
        puts = []
        for b in range(GK):
            gets[b].wait()
            j = j0 * GK + b
            puts.append(pltpu.async_copy(rows_v.at[b], out.at[pl.ds(base + j * GB, GB)], sem_s))
        for b in range(GK):
            puts[b].wait()
        return 0

    lax.fori_loop(0, GC // GK, outer, 0)


def _sc_gather(table, idx3):
    return pl.kernel(
        _gather_body,
        out_type=jax.ShapeDtypeStruct((2 * E, D), jnp.float32),
        mesh=plsc.VectorSubcoreMesh(**_MESH),
        scratch_types=[
            pltpu.VMEM((GC, GB), jnp.int32),
            pltpu.VMEM((GK, GB, D), jnp.float32),
            pltpu.SemaphoreType.DMA,
            pltpu.SemaphoreType.DMA,
        ],
    )(table, idx3)


def _scatter_body(e2, dst_hbm, zeros_hbm, out, dst_v, e_v, sem, agg_sh):
    c = lax.axis_index("c")
    s = lax.axis_index("s")
    wid = s * NC + c
    pltpu.sync_copy(zeros_hbm.at[pl.ds(0, ZB)], agg_sh.at[pl.ds(s * ZB, ZB)])

    @pl.when(s == 0)
    def _zero_tail():
        pltpu.sync_copy(zeros_hbm.at[pl.ds(0, ZTAIL)],
                        agg_sh.at[pl.ds(NS * ZB, ZTAIL)])

    pltpu.sync_copy(dst_hbm.at[wid], dst_v)
    plsc.subcore_barrier()
    base = wid * S_PER

    def outer(j0, _):
        gets = []
        for b in range(SK):
            j = j0 * SK + b
            gets.append(pltpu.async_copy(e2.at[pl.ds(base + j * SB, SB)], e_v.at[b], sem))
        for b in range(SK):
            gets[b].wait()
            j = j0 * SK + b
            pltpu.sync_copy(e_v.at[b], agg_sh.at[dst_v.at[j]], add=True)
        return 0

    lax.fori_loop(0, SCH // SK, outer, 0)
    plsc.subcore_barrier()
    pltpu.sync_copy(agg_sh.at[pl.ds(s * ZB, ZB)],
                    out.at[c].at[pl.ds(s * ZB, ZB)])

    @pl.when(s == 0)
    def _read_tail():
        pltpu.sync_copy(agg_sh.at[pl.ds(NS * ZB, ZTAIL)],
                        out.at[c].at[pl.ds(NS * ZB, ZTAIL)])


def _sc_scatter(e2, dst3, zeros):
    return pl.kernel(
        _scatter_body,
        out_type=jax.ShapeDtypeStruct((NC, N, D), jnp.float32),
        mesh=plsc.VectorSubcoreMesh(**_MESH),
        scratch_types=[
            pltpu.VMEM((SCH, SB), jnp.int32),
            pltpu.VMEM((SK, SB, D), jnp.float32),
            pltpu.SemaphoreType.DMA,
            pltpu.VMEM_SHARED((N, D), jnp.float32),
        ],
    )(e2, dst3, zeros)


def _silu(v):
    return v * (1.0 / (1.0 + jnp.exp(-v)))


def _mlp_tail(h3, g, beta):
    mu = jnp.mean(h3, axis=-1, keepdims=True)
    dlt = h3 - mu
    var = jnp.mean(dlt * dlt, axis=-1, keepdims=True)
    return dlt * lax.rsqrt(var + 1e-5) * g + beta


def _edge_block(e_ref, gs_ref, gd_ref, w1e, w2, w3, b1, b2, b3, g, beta, out_ref):
    e = e_ref[...]
    pre = jnp.dot(e, w1e[...], preferred_element_type=jnp.float32)
    pre = pre + gs_ref[...] + gd_ref[...] + b1[...]
    h1 = _silu(pre)
    h2 = _silu(jnp.dot(h1, w2[...], preferred_element_type=jnp.float32) + b2[...])
    h3 = jnp.dot(h2, w3[...], preferred_element_type=jnp.float32) + b3[...]
    out_ref[...] = _mlp_tail(h3, g[...], beta[...]) + e


BE = 2000


def _tc_edge(e, gs, gd, w1e, w2, w3, b1, b2, b3, g, beta):
    row = lambda i: (i, 0)
    fix = lambda i: (0, 0)
    return pl.pallas_call(
        _edge_block,
        grid=(E // BE,),
        in_specs=[
            pl.BlockSpec((BE, D), row),
            pl.BlockSpec((BE, D), row),
            pl.BlockSpec((BE, D), row),
            pl.BlockSpec((D, D), fix),
            pl.BlockSpec((D, D), fix),
            pl.BlockSpec((D, D), fix),
            pl.BlockSpec((1, D), fix),
            pl.BlockSpec((1, D), fix),
            pl.BlockSpec((1, D), fix),
            pl.BlockSpec((1, D), fix),
            pl.BlockSpec((1, D), fix),
        ],
        out_specs=pl.BlockSpec((BE, D), row),
        out_shape=jax.ShapeDtypeStruct((E, D), jnp.float32),
    )(e, gs, gd, w1e, w2, w3, b1, b2, b3, g, beta)


def _node_block(agg2_ref, x_ref, w1a, w1x, w2, w3, b1, b2, b3, g, beta,
                wys, wyd, xo_ref, y_ref):
    x = x_ref[...]
    agg = agg2_ref[0] + agg2_ref[1]
    pre = (jnp.dot(agg, w1a[...], preferred_element_type=jnp.float32)
           + jnp.dot(x, w1x[...], preferred_element_type=jnp.float32) + b1[...])
    h1 = _silu(pre)
    h2 = _silu(jnp.dot(h1, w2[...], preferred_element_type=jnp.float32) + b2[...])
    h3 = jnp.dot(h2, w3[...], preferred_element_type=jnp.float32) + b3[...]
    xo = _mlp_tail(h3, g[...], beta[...]) + x
    xo_ref[...] = xo
    y_ref[0] = jnp.dot(xo, wys[...], preferred_element_type=jnp.float32)
    y_ref[1] = jnp.dot(xo, wyd[...], preferred_element_type=jnp.float32)


BN = 2000


def _tc_node(agg2, x, w1a, w1x, w2, w3, b1, b2, b3, g, beta, wys, wyd):
    row = lambda i: (i, 0)
    fix = lambda i: (0, 0)
    pair = lambda i: (0, i, 0)
    return pl.pallas_call(
        _node_block,
        grid=(N // BN,),
        in_specs=[
            pl.BlockSpec((2, BN, D), pair),
            pl.BlockSpec((BN, D), row),
            pl.BlockSpec((D, D), fix),
            pl.BlockSpec((D, D), fix),
            pl.BlockSpec((D, D), fix),
            pl.BlockSpec((D, D), fix),
            pl.BlockSpec((1, D), fix),
            pl.BlockSpec((1, D), fix),
            pl.BlockSpec((1, D), fix),
            pl.BlockSpec((1, D), fix),
            pl.BlockSpec((1, D), fix),
            pl.BlockSpec((D, D), fix),
            pl.BlockSpec((D, D), fix),
        ],
        out_specs=[pl.BlockSpec((BN, D), row), pl.BlockSpec((2, BN, D), pair)],
        out_shape=[jax.ShapeDtypeStruct((N, D), jnp.float32),
                   jax.ShapeDtypeStruct((2, N, D), jnp.float32)],
    )(agg2, x, w1a, w1x, w2, w3, b1, b2, b3, g, beta, wys, wyd)


def _init_block(x_ref, wys, wyd, y_ref):
    x = x_ref[...]
    y_ref[0] = jnp.dot(x, wys[...], preferred_element_type=jnp.float32)
    y_ref[1] = jnp.dot(x, wyd[...], preferred_element_type=jnp.float32)


def _tc_init(x, wys, wyd):
    return pl.pallas_call(
        _init_block,
        grid=(N // BN,),
        in_specs=[
            pl.BlockSpec((BN, D), lambda i: (i, 0)),
            pl.BlockSpec((D, D), lambda i: (0, 0)),
            pl.BlockSpec((D, D), lambda i: (0, 0)),
        ],
        out_specs=pl.BlockSpec((2, BN, D), lambda i: (0, i, 0)),
        out_shape=jax.ShapeDtypeStruct((2, N, D), jnp.float32),
    )(x, wys, wyd)


def kernel(node_features, edge_features, edge_index, eW1, eb1, eW2, eb2,
           eW3, eb3, eg, ebeta, nW1, nb1, nW2, nb2, nW3, nb3, ng, nbeta):
    src = edge_index[0]
    dst = edge_index[1]
    idx3 = jnp.concatenate([src, dst + N]).reshape(NW, GC, GB)
    dst3 = dst.reshape(NW, SCH, SB)
    zeros = jnp.zeros((ZB, D), jnp.float32)

    W1e = eW1[:, :D]
    W1s = eW1[:, D:2 * D]
    W1d = eW1[:, 2 * D:]
    nW1a = nW1[:, :D]
    nW1x = nW1[:, D:]
    r1 = lambda v: v.reshape(1, D)

    x = node_features
    e = edge_features
    y = _tc_init(x, W1s[0], W1d[0])
    for i in range(P):
        g2 = _sc_gather(y.reshape(2 * N, D), idx3)
        e = _tc_edge(e, g2[:E], g2[E:], W1e[i], eW2[i], eW3[i],
                     r1(eb1[i]), r1(eb2[i]), r1(eb3[i]), r1(eg[i]), r1(ebeta[i]))
        agg2 = _sc_scatter(e, dst3, zeros)
        j = min(i + 1, P - 1)
        x, y = _tc_node(agg2, x, nW1a[i], nW1x[i], nW2[i], nW3[i],
                        r1(nb1[i]), r1(nb2[i]), r1(nb3[i]), r1(ng[i]), r1(nbeta[i]),
                        W1s[j], W1d[j])
    return x


# speedup vs baseline: 4.3552x; 1.4113x over previous
import functools

import jax
import jax.numpy as jnp
from jax import lax
from jax.experimental import pallas as pl
from jax.experimental.pallas import tpu as pltpu
from jax.experimental.pallas import tpu_sc as plsc

P = 10
D = 128
N = 10000
E = 160000

NC = 2
NS = 16
NW = NC * NS

G_PER = 2 * E // NW
GB = 80
GC = G_PER // GB
GK = 5

S_PER = E // NW
SB = 40
SCH = S_PER // SB
SK = 5
ZB = 624
ZTAIL = N - NS * ZB

_MESH = dict(core_axis_name="c", subcore_axis_name="s")


def _gather_body(table, idx_hbm, out, idx_v, rows_v, sem_g, sem_s):
    wid = lax.axis_index("s") * NC + lax.axis_index("c")
    pltpu.sync_copy(idx_hbm.at[wid], idx_v)
    base = wid * G_PER
    nout = GC // GK

    def fire_g(grp, half):
        for b in range(GK):
            pltpu.async_copy(table.at[idx_v.at[grp * GK + b]], rows_v.at[half, b], sem_g)

    def wait_g(grp, half):
        for b in range(GK):
            pltpu.make_async_copy(table.at[idx_v.at[grp * GK + b]], rows_v.at[half, b], sem_g).wait()

    def fire_s(grp, half):
        for b in range(GK):
            j = grp * GK + b
            pltpu.async_copy(rows_v.at[half, b], out.at[pl.ds(base + j * GB, GB)], sem_s)

    def wait_s(grp, half):
        for b in range(GK):
            j = grp * GK + b
            pltpu.make_async_copy(rows_v.at[half, b], out.at[pl.ds(base + j * GB, GB)], sem_s).wait()

    fire_g(0, 0)

    def body(j0, _):
        p = lax.rem(j0, 2)
        q = 1 - p

        @pl.when(j0 > 0)
        def _drain_prev_stores():
            wait_s(j0 - 1, q)

        @pl.when(j0 + 1 < nout)
        def _fire_next_gathers():
            fire_g(j0 + 1, q)

        wait_g(j0, p)
        fire_s(j0, p)
        return 0

    lax.fori_loop(0, nout, body, 0)
    wait_s(nout - 1, (nout - 1) % 2)


def _sc_gather(table, idx3):
    return pl.kernel(
        _gather_body,
        out_type=jax.ShapeDtypeStruct((2 * E, D), jnp.float32),
        mesh=plsc.VectorSubcoreMesh(**_MESH),
        scratch_types=[
            pltpu.VMEM((GC, GB), jnp.int32),
            pltpu.VMEM((2, GK, GB, D), jnp.float32),
            pltpu.SemaphoreType.DMA,
            pltpu.SemaphoreType.DMA,
        ],
    )(table, idx3)


def _scatter_body(e2, dst_hbm, zeros_hbm, out, dst_v, e_v, sem, sem_a, agg_sh):
    c = lax.axis_index("c")
    s = lax.axis_index("s")
    wid = s * NC + c
    pltpu.sync_copy(zeros_hbm.at[pl.ds(0, ZB)], agg_sh.at[pl.ds(s * ZB, ZB)])

    @pl.when(s == 0)
    def _zero_tail():
        pltpu.sync_copy(zeros_hbm.at[pl.ds(0, ZTAIL)],
                        agg_sh.at[pl.ds(NS * ZB, ZTAIL)])

    pltpu.sync_copy(dst_hbm.at[wid], dst_v)
    plsc.subcore_barrier()
    base = wid * S_PER
    nout = SCH // SK

    def body(j0, _):
        gets = []
        for b in range(SK):
            j = j0 * SK + b
            gets.append(pltpu.async_copy(e2.at[pl.ds(base + j * SB, SB)], e_v.at[b], sem))
        adds = []
        for b in range(SK):
            gets[b].wait()
            adds.append(pltpu.async_copy(e_v.at[b], agg_sh.at[dst_v.at[j0 * SK + b]],
                                         sem_a, add=True))
        for b in range(SK):
            adds[b].wait()
        return 0

    lax.fori_loop(0, nout, body, 0)
    plsc.subcore_barrier()
    pltpu.sync_copy(agg_sh.at[pl.ds(s * ZB, ZB)],
                    out.at[c].at[pl.ds(s * ZB, ZB)])

    @pl.when(s == 0)
    def _read_tail():
        pltpu.sync_copy(agg_sh.at[pl.ds(NS * ZB, ZTAIL)],
                        out.at[c].at[pl.ds(NS * ZB, ZTAIL)])


def _sc_scatter(e2, dst3, zeros):
    return pl.kernel(
        _scatter_body,
        out_type=jax.ShapeDtypeStruct((NC, N, D), jnp.float32),
        mesh=plsc.VectorSubcoreMesh(**_MESH),
        scratch_types=[
            pltpu.VMEM((SCH, SB), jnp.int32),
            pltpu.VMEM((SK, SB, D), jnp.float32),
            pltpu.SemaphoreType.DMA,
            pltpu.SemaphoreType.DMA,
            pltpu.VMEM_SHARED((N, D), jnp.float32),
        ],
    )(e2, dst3, zeros)


def _silu(v):
    return v * (1.0 / (1.0 + jnp.exp(-v)))


def _mlp_tail(h3, g, beta):
    mu = jnp.mean(h3, axis=-1, keepdims=True)
    dlt = h3 - mu
    var = jnp.mean(dlt * dlt, axis=-1, keepdims=True)
    return dlt * lax.rsqrt(var + 1e-5) * g + beta


def _edge_block(e_ref, gs_ref, gd_ref, w1e, w2, w3, b1, b2, b3, g, beta, out_ref):
    e = e_ref[...]
    pre = jnp.dot(e, w1e[...], preferred_element_type=jnp.float32)
    pre = pre + gs_ref[...] + gd_ref[...] + b1[...]
    h1 = _silu(pre)
    h2 = _silu(jnp.dot(h1, w2[...], preferred_element_type=jnp.float32) + b2[...])
    h3 = jnp.dot(h2, w3[...], preferred_element_type=jnp.float32) + b3[...]
    out_ref[...] = _mlp_tail(h3, g[...], beta[...]) + e


BE = 4000


def _tc_edge(e, g2, w1e, w2, w3, b1, b2, b3, g, beta):
    row = lambda i: (i, 0)
    fix = lambda i: (0, 0)
    return pl.pallas_call(
        _edge_block,
        grid=(E // BE,),
        in_specs=[
            pl.BlockSpec((BE, D), row),
            pl.BlockSpec((BE, D), row),
            pl.BlockSpec((BE, D), lambda i: (i + E // BE, 0)),
            pl.BlockSpec((D, D), fix),
            pl.BlockSpec((D, D), fix),
            pl.BlockSpec((D, D), fix),
            pl.BlockSpec((1, D), fix),
            pl.BlockSpec((1, D), fix),
            pl.BlockSpec((1, D), fix),
            pl.BlockSpec((1, D), fix),
            pl.BlockSpec((1, D), fix),
        ],
        out_specs=pl.BlockSpec((BE, D), row),
        out_shape=jax.ShapeDtypeStruct((E, D), jnp.float32),
    )(e, g2, g2, w1e, w2, w3, b1, b2, b3, g, beta)


def _node_block(agg2_ref, x_ref, w1a, w1x, w2, w3, b1, b2, b3, g, beta,
                wys, wyd, xo_ref, y_ref):
    x = x_ref[...]
    agg = agg2_ref[0] + agg2_ref[1]
    pre = (jnp.dot(agg, w1a[...], preferred_element_type=jnp.float32)
           + jnp.dot(x, w1x[...], preferred_element_type=jnp.float32) + b1[...])
    h1 = _silu(pre)
    h2 = _silu(jnp.dot(h1, w2[...], preferred_element_type=jnp.float32) + b2[...])
    h3 = jnp.dot(h2, w3[...], preferred_element_type=jnp.float32) + b3[...]
    xo = _mlp_tail(h3, g[...], beta[...]) + x
    xo_ref[...] = xo
    y_ref[0] = jnp.dot(xo, wys[...], preferred_element_type=jnp.float32)
    y_ref[1] = jnp.dot(xo, wyd[...], preferred_element_type=jnp.float32)


BN = 2000


def _tc_node(agg2, x, w1a, w1x, w2, w3, b1, b2, b3, g, beta, wys, wyd):
    row = lambda i: (i, 0)
    fix = lambda i: (0, 0)
    pair = lambda i: (0, i, 0)
    return pl.pallas_call(
        _node_block,
        grid=(N // BN,),
        in_specs=[
            pl.BlockSpec((2, BN, D), pair),
            pl.BlockSpec((BN, D), row),
            pl.BlockSpec((D, D), fix),
            pl.BlockSpec((D, D), fix),
            pl.BlockSpec((D, D), fix),
            pl.BlockSpec((D, D), fix),
            pl.BlockSpec((1, D), fix),
            pl.BlockSpec((1, D), fix),
            pl.BlockSpec((1, D), fix),
            pl.BlockSpec((1, D), fix),
            pl.BlockSpec((1, D), fix),
            pl.BlockSpec((D, D), fix),
            pl.BlockSpec((D, D), fix),
        ],
        out_specs=[pl.BlockSpec((BN, D), row), pl.BlockSpec((2, BN, D), pair)],
        out_shape=[jax.ShapeDtypeStruct((N, D), jnp.float32),
                   jax.ShapeDtypeStruct((2, N, D), jnp.float32)],
    )(agg2, x, w1a, w1x, w2, w3, b1, b2, b3, g, beta, wys, wyd)


def _init_block(x_ref, wys, wyd, y_ref):
    x = x_ref[...]
    y_ref[0] = jnp.dot(x, wys[...], preferred_element_type=jnp.float32)
    y_ref[1] = jnp.dot(x, wyd[...], preferred_element_type=jnp.float32)


def _tc_init(x, wys, wyd):
    return pl.pallas_call(
        _init_block,
        grid=(N // BN,),
        in_specs=[
            pl.BlockSpec((BN, D), lambda i: (i, 0)),
            pl.BlockSpec((D, D), lambda i: (0, 0)),
            pl.BlockSpec((D, D), lambda i: (0, 0)),
        ],
        out_specs=pl.BlockSpec((2, BN, D), lambda i: (0, i, 0)),
        out_shape=jax.ShapeDtypeStruct((2, N, D), jnp.float32),
    )(x, wys, wyd)


def kernel(node_features, edge_features, edge_index, eW1, eb1, eW2, eb2,
           eW3, eb3, eg, ebeta, nW1, nb1, nW2, nb2, nW3, nb3, ng, nbeta):
    src = edge_index[0]
    dst = edge_index[1]
    idx3 = jnp.concatenate([src, dst + N]).reshape(NW, GC, GB)
    dst3 = dst.reshape(NW, SCH, SB)
    zeros = jnp.zeros((ZB, D), jnp.float32)

    W1e = eW1[:, :D]
    W1s = eW1[:, D:2 * D]
    W1d = eW1[:, 2 * D:]
    nW1a = nW1[:, :D]
    nW1x = nW1[:, D:]
    r1 = lambda v: v.reshape(1, D)

    x = node_features
    e = edge_features
    y = _tc_init(x, W1s[0], W1d[0])
    for i in range(P):
        g2 = _sc_gather(y.reshape(2 * N, D), idx3)
        e = _tc_edge(e, g2, W1e[i], eW2[i], eW3[i],
                     r1(eb1[i]), r1(eb2[i]), r1(eb3[i]), r1(eg[i]), r1(ebeta[i]))
        agg2 = _sc_scatter(e, dst3, zeros)
        j = min(i + 1, P - 1)
        x, y = _tc_node(agg2, x, nW1a[i], nW1x[i], nW2[i], nW3[i],
                        r1(nb1[i]), r1(nb2[i]), r1(nb3[i]), r1(ng[i]), r1(nbeta[i]),
                        W1s[j], W1d[j])
    return x


# speedup vs baseline: 4.3775x; 1.0051x over previous
import functools

import jax
import jax.numpy as jnp
from jax import lax
from jax.experimental import pallas as pl
from jax.experimental.pallas import tpu as pltpu
from jax.experimental.pallas import tpu_sc as plsc

P = 10
D = 128
N = 10000
E = 160000

NC = 2
NS = 16
NW = NC * NS

EH = E // 2
G_PER = 2 * EH // NW
GB = 40
GC = G_PER // GB
GK = 5

S_PER = E // NW
SB = 40
SCH = S_PER // SB
SK = 5
ZB = 624
ZTAIL = N - NS * ZB

_MESH = dict(core_axis_name="c", subcore_axis_name="s")


def _gather_body(table, idx_hbm, out, idx_v, rows_v, sem_g, sem_s):
    wid = lax.axis_index("s") * NC + lax.axis_index("c")
    pltpu.sync_copy(idx_hbm.at[wid], idx_v)
    base = wid * G_PER
    nout = GC // GK

    def fire_g(grp, half):
        for b in range(GK):
            pltpu.async_copy(table.at[idx_v.at[grp * GK + b]], rows_v.at[half, b], sem_g)

    def wait_g(grp, half):
        for b in range(GK):
            pltpu.make_async_copy(table.at[idx_v.at[grp * GK + b]], rows_v.at[half, b], sem_g).wait()

    def fire_s(grp, half):
        for b in range(GK):
            j = grp * GK + b
            pltpu.async_copy(rows_v.at[half, b], out.at[pl.ds(base + j * GB, GB)], sem_s)

    def wait_s(grp, half):
        for b in range(GK):
            j = grp * GK + b
            pltpu.make_async_copy(rows_v.at[half, b], out.at[pl.ds(base + j * GB, GB)], sem_s).wait()

    fire_g(0, 0)

    def body(j0, _):
        p = lax.rem(j0, 2)
        q = 1 - p

        @pl.when(j0 > 0)
        def _drain_prev_stores():
            wait_s(j0 - 1, q)

        @pl.when(j0 + 1 < nout)
        def _fire_next_gathers():
            fire_g(j0 + 1, q)

        wait_g(j0, p)
        fire_s(j0, p)
        return 0

    lax.fori_loop(0, nout, body, 0)
    wait_s(nout - 1, (nout - 1) % 2)


def _sc_gather(table, idx3):
    return pl.kernel(
        _gather_body,
        out_type=jax.ShapeDtypeStruct((2 * EH, D), jnp.float32),
        mesh=plsc.VectorSubcoreMesh(**_MESH),
        scratch_types=[
            pltpu.VMEM((GC, GB), jnp.int32),
            pltpu.VMEM((2, GK, GB, D), jnp.float32),
            pltpu.SemaphoreType.DMA,
            pltpu.SemaphoreType.DMA,
        ],
    )(table, idx3)


def _scatter_body(e2, dst_hbm, zeros_hbm, out, dst_v, e_v, sem, sem_a, agg_sh):
    c = lax.axis_index("c")
    s = lax.axis_index("s")
    wid = s * NC + c
    pltpu.sync_copy(zeros_hbm.at[pl.ds(0, ZB)], agg_sh.at[pl.ds(s * ZB, ZB)])

    @pl.when(s == 0)
    def _zero_tail():
        pltpu.sync_copy(zeros_hbm.at[pl.ds(0, ZTAIL)],
                        agg_sh.at[pl.ds(NS * ZB, ZTAIL)])

    pltpu.sync_copy(dst_hbm.at[wid], dst_v)
    plsc.subcore_barrier()
    base = wid * S_PER
    nout = SCH // SK

    def body(j0, _):
        gets = []
        for b in range(SK):
            j = j0 * SK + b
            gets.append(pltpu.async_copy(e2.at[pl.ds(base + j * SB, SB)], e_v.at[b], sem))
        adds = []
        for b in range(SK):
            gets[b].wait()
            adds.append(pltpu.async_copy(e_v.at[b], agg_sh.at[dst_v.at[j0 * SK + b]],
                                         sem_a, add=True))
        for b in range(SK):
            adds[b].wait()
        return 0

    lax.fori_loop(0, nout, body, 0)
    plsc.subcore_barrier()
    pltpu.sync_copy(agg_sh.at[pl.ds(s * ZB, ZB)],
                    out.at[c].at[pl.ds(s * ZB, ZB)])

    @pl.when(s == 0)
    def _read_tail():
        pltpu.sync_copy(agg_sh.at[pl.ds(NS * ZB, ZTAIL)],
                        out.at[c].at[pl.ds(NS * ZB, ZTAIL)])


def _sc_scatter(e2, dst3, zeros):
    return pl.kernel(
        _scatter_body,
        out_type=jax.ShapeDtypeStruct((NC, N, D), jnp.float32),
        mesh=plsc.VectorSubcoreMesh(**_MESH),
        scratch_types=[
            pltpu.VMEM((SCH, SB), jnp.int32),
            pltpu.VMEM((SK, SB, D), jnp.float32),
            pltpu.SemaphoreType.DMA,
            pltpu.SemaphoreType.DMA,
            pltpu.VMEM_SHARED((N, D), jnp.float32),
        ],
    )(e2, dst3, zeros)


def _silu(v):
    return v * (1.0 / (1.0 + jnp.exp(-v)))


def _mlp_tail(h3, g, beta):
    mu = jnp.mean(h3, axis=-1, keepdims=True)
    dlt = h3 - mu
    var = jnp.mean(dlt * dlt, axis=-1, keepdims=True)
    return dlt * lax.rsqrt(var + 1e-5) * g + beta


def _edge_block(e_ref, gs_ref, gd_ref, w1e, w2, w3, b1, b2, b3, g, beta, out_ref):
    e = e_ref[...]
    pre = jnp.dot(e, w1e[...], preferred_element_type=jnp.float32)
    pre = pre + gs_ref[...] + gd_ref[...] + b1[...]
    h1 = _silu(pre)
    h2 = _silu(jnp.dot(h1, w2[...], preferred_element_type=jnp.float32) + b2[...])
    h3 = jnp.dot(h2, w3[...], preferred_element_type=jnp.float32) + b3[...]
    out_ref[...] = _mlp_tail(h3, g[...], beta[...]) + e


BE = 4000
NBH = EH // BE


def _edge_block_alias(e_ref, gs_ref, gd_ref, prev_ref, w1e, w2, w3,
                      b1, b2, b3, g, beta, out_ref):
    del prev_ref
    _edge_block(e_ref, gs_ref, gd_ref, w1e, w2, w3, b1, b2, b3, g, beta, out_ref)


def _tc_edge_half(e, g2, prev, off, w1e, w2, w3, b1, b2, b3, g, beta):
    fix = lambda i: (0, 0)
    row = lambda i: (i + off, 0)
    specs = [
        pl.BlockSpec((BE, D), row),
        pl.BlockSpec((BE, D), lambda i: (i, 0)),
        pl.BlockSpec((BE, D), lambda i: (i + NBH, 0)),
    ]
    args = [e, g2, g2]
    body = _edge_block
    aliases = {}
    if prev is not None:
        specs.append(pl.BlockSpec(memory_space=pl.ANY))
        args.append(prev)
        body = _edge_block_alias
        aliases = {3: 0}
    specs += [pl.BlockSpec((D, D), fix)] * 3 + [pl.BlockSpec((1, D), fix)] * 5
    args += [w1e, w2, w3, b1, b2, b3, g, beta]
    return pl.pallas_call(
        body,
        grid=(NBH,),
        in_specs=specs,
        out_specs=pl.BlockSpec((BE, D), row),
        out_shape=jax.ShapeDtypeStruct((E, D), jnp.float32),
        input_output_aliases=aliases,
    )(*args)


def _node_block(agg2_ref, x_ref, w1a, w1x, w2, w3, b1, b2, b3, g, beta,
                wys, wyd, xo_ref, y_ref):
    x = x_ref[...]
    agg = agg2_ref[0] + agg2_ref[1]
    pre = (jnp.dot(agg, w1a[...], preferred_element_type=jnp.float32)
           + jnp.dot(x, w1x[...], preferred_element_type=jnp.float32) + b1[...])
    h1 = _silu(pre)
    h2 = _silu(jnp.dot(h1, w2[...], preferred_element_type=jnp.float32) + b2[...])
    h3 = jnp.dot(h2, w3[...], preferred_element_type=jnp.float32) + b3[...]
    xo = _mlp_tail(h3, g[...], beta[...]) + x
    xo_ref[...] = xo
    y_ref[0] = jnp.dot(xo, wys[...], preferred_element_type=jnp.float32)
    y_ref[1] = jnp.dot(xo, wyd[...], preferred_element_type=jnp.float32)


BN = 2000


def _tc_node(agg2, x, w1a, w1x, w2, w3, b1, b2, b3, g, beta, wys, wyd):
    row = lambda i: (i, 0)
    fix = lambda i: (0, 0)
    pair = lambda i: (0, i, 0)
    return pl.pallas_call(
        _node_block,
        grid=(N // BN,),
        in_specs=[
            pl.BlockSpec((2, BN, D), pair),
            pl.BlockSpec((BN, D), row),
            pl.BlockSpec((D, D), fix),
            pl.BlockSpec((D, D), fix),
            pl.BlockSpec((D, D), fix),
            pl.BlockSpec((D, D), fix),
            pl.BlockSpec((1, D), fix),
            pl.BlockSpec((1, D), fix),
            pl.BlockSpec((1, D), fix),
            pl.BlockSpec((1, D), fix),
            pl.BlockSpec((1, D), fix),
            pl.BlockSpec((D, D), fix),
            pl.BlockSpec((D, D), fix),
        ],
        out_specs=[pl.BlockSpec((BN, D), row), pl.BlockSpec((2, BN, D), pair)],
        out_shape=[jax.ShapeDtypeStruct((N, D), jnp.float32),
                   jax.ShapeDtypeStruct((2, N, D), jnp.float32)],
    )(agg2, x, w1a, w1x, w2, w3, b1, b2, b3, g, beta, wys, wyd)


def _init_block(x_ref, wys, wyd, y_ref):
    x = x_ref[...]
    y_ref[0] = jnp.dot(x, wys[...], preferred_element_type=jnp.float32)
    y_ref[1] = jnp.dot(x, wyd[...], preferred_element_type=jnp.float32)


def _tc_init(x, wys, wyd):
    return pl.pallas_call(
        _init_block,
        grid=(N // BN,),
        in_specs=[
            pl.BlockSpec((BN, D), lambda i: (i, 0)),
            pl.BlockSpec((D, D), lambda i: (0, 0)),
            pl.BlockSpec((D, D), lambda i: (0, 0)),
        ],
        out_specs=pl.BlockSpec((2, BN, D), lambda i: (0, i, 0)),
        out_shape=jax.ShapeDtypeStruct((2, N, D), jnp.float32),
    )(x, wys, wyd)


def kernel(node_features, edge_features, edge_index, eW1, eb1, eW2, eb2,
           eW3, eb3, eg, ebeta, nW1, nb1, nW2, nb2, nW3, nb3, ng, nbeta):
    src = edge_index[0]
    dst = edge_index[1]
    idxA = jnp.concatenate([src[:EH], dst[:EH] + N]).reshape(NW, GC, GB)
    idxB = jnp.concatenate([src[EH:], dst[EH:] + N]).reshape(NW, GC, GB)
    dst3 = dst.reshape(NW, SCH, SB)
    zeros = jnp.zeros((ZB, D), jnp.float32)

    W1e = eW1[:, :D]
    W1s = eW1[:, D:2 * D]
    W1d = eW1[:, 2 * D:]
    nW1a = nW1[:, :D]
    nW1x = nW1[:, D:]
    r1 = lambda v: v.reshape(1, D)

    x = node_features
    e = edge_features
    y = _tc_init(x, W1s[0], W1d[0])
    for i in range(P):
        table = y.reshape(2 * N, D)
        ew = (W1e[i], eW2[i], eW3[i],
              r1(eb1[i]), r1(eb2[i]), r1(eb3[i]), r1(eg[i]), r1(ebeta[i]))
        gA = _sc_gather(table, idxA)
        gB = _sc_gather(table, idxB)
        eA = _tc_edge_half(e, gA, None, 0, *ew)
        e = _tc_edge_half(e, gB, eA, NBH, *ew)
        agg2 = _sc_scatter(e, dst3, zeros)
        j = min(i + 1, P - 1)
        x, y = _tc_node(agg2, x, nW1a[i], nW1x[i], nW2[i], nW3[i],
                        r1(nb1[i]), r1(nb2[i]), r1(nb3[i]), r1(ng[i]), r1(nbeta[i]),
                        W1s[j], W1d[j])
    return x
